# baseline (speedup 1.0000x reference)
import functools

import jax
import jax.numpy as jnp
from jax import lax
from jax.experimental import pallas as pl
from jax.experimental.pallas import tpu as pltpu
from jax.experimental.pallas import tpu_sc as plsc

_NC = 2
_NS = 16
_NW = _NC * _NS
_L = 16

_LN2 = 0.6931471805599453
_LOGC = (-1.9316715417209647, 3.498227901209959, -2.420812563219248,
         1.1048082361995168, -0.2806325404497544, 0.030102625011692218)


def _vf(v):
    return jnp.full((_L,), v, jnp.float32)


def _focal_partials(pred_flat, tgt, aux, rows_per_worker, chunk_rows):
    nch = rows_per_worker // chunk_rows
    niter = chunk_rows // _L
    mesh = plsc.VectorSubcoreMesh(core_axis_name="c", subcore_axis_name="s")

    @functools.partial(
        pl.kernel,
        out_type=jax.ShapeDtypeStruct((_NW * _L,), jnp.float32),
        mesh=mesh,
        scratch_types=[
            pltpu.VMEM((2 * chunk_rows,), jnp.float32),
            pltpu.VMEM((2 * chunk_rows,), jnp.float32),
            pltpu.VMEM((chunk_rows,), jnp.int32),
            pltpu.VMEM((chunk_rows,), jnp.int32),
            pltpu.VMEM((2 * _L,), jnp.float32),
            pltpu.VMEM((_L,), jnp.float32),
            pltpu.SemaphoreType.DMA,
            pltpu.SemaphoreType.DMA,
            pltpu.SemaphoreType.DMA,
            pltpu.SemaphoreType.DMA,
        ],
        compiler_params=pltpu.CompilerParams(needs_layout_passes=False),
    )
    def k(pred_hbm, tgt_hbm, aux_hbm, out_hbm,
          pb0, pb1, tb0, tb1, auxv, accv, sp0, sp1, st0, st1):
        wid = lax.axis_index("s") * _NC + lax.axis_index("c")
        pbase = pl.multiple_of(wid * (2 * rows_per_worker), 8)
        tbase = pl.multiple_of(wid * rows_per_worker, 8)

        pltpu.sync_copy(aux_hbm, auxv)
        a0 = auxv[pl.ds(0, _L)]
        ad = auxv[pl.ds(_L, _L)]

        pbufs = (pb0, pb1)
        tbufs = (tb0, tb1)
        psems = (sp0, sp1)
        tsems = (st0, st1)
        copies = [None, None]

        def start(g, b):
            cp = pltpu.async_copy(
                pred_hbm.at[pl.ds(pbase + g * (2 * chunk_rows), 2 * chunk_rows)],
                pbufs[b], psems[b])
            ct = pltpu.async_copy(
                tgt_hbm.at[pl.ds(tbase + g * chunk_rows, chunk_rows)],
                tbufs[b], tsems[b])
            copies[b] = (cp, ct)

        start(0, 0)
        if nch > 1:
            start(1, 1)

        idx_init = lax.iota(jnp.int32, _L) * 2
        ones_i = jnp.full((_L,), 1, jnp.int32)
        step_i = jnp.full((_L,), 2 * _L, jnp.int32)
        acc = jnp.zeros((_L,), jnp.float32)

        c5 = _vf(_LOGC[5])
        c4 = _vf(_LOGC[4])
        c3 = _vf(_LOGC[3])
        c2 = _vf(_LOGC[2])
        c1 = _vf(_LOGC[1])
        c0 = _vf(_LOGC[0])
        one = _vf(1.0)
        clamp = _vf(80.0)
        ln2 = _vf(_LN2)
        e_bias = _vf(127.0)
        mant_mask = jnp.full((_L,), 0x007FFFFF, jnp.int32)
        one_bits = jnp.full((_L,), 0x3F800000, jnp.int32)
        shift23 = jnp.full((_L,), 23, jnp.int32)

        for g in range(nch):
            b = g & 1
            cp, ct = copies[b]
            cp.wait()
            ct.wait()
            pbuf = pbufs[b]
            tbuf = tbufs[b]

            def body(j, carry, pbuf=pbuf, tbuf=tbuf):
                acc, idx0 = carry
                x0 = plsc.load_gather(pbuf, [idx0])
                x1 = plsc.load_gather(pbuf, [idx0 + ones_i])
                tv = tbuf[pl.ds(j * _L, _L)]
                tf = tv.astype(jnp.float32)
                s2 = tf + tf - one
                nz = (x0 - x1) * s2
                nz = jnp.minimum(nz, clamp)
                u = jnp.exp(nz)
                w = u + one
                r = one / w
                q = u * r
                sq = q * q
                bits = plsc.bitcast(w, jnp.int32)
                e = lax.shift_right_logical(bits, shift23)
                mbits = jnp.bitwise_or(jnp.bitwise_and(bits, mant_mask),
                                       one_bits)
                mm = plsc.bitcast(mbits, jnp.float32)
                pol = c5
                pol = pol * mm + c4
                pol = pol * mm + c3
                pol = pol * mm + c2
                pol = pol * mm + c1
                pol = pol * mm + c0
                ef = e.astype(jnp.float32) - e_bias
                logw = ef * ln2 + pol
                at = a0 + tf * ad
                acc = acc + at * (sq * logw)
                return (acc, idx0 + step_i)

            acc, _ = lax.fori_loop(0, niter, body, (acc, idx_init))
            if g + 2 < nch:
                start(g + 2, b)

        accv[...] = acc
        pltpu.sync_copy(accv, out_hbm.at[pl.ds(pl.multiple_of(wid * _L, 8), _L)])

    return k(pred_flat, tgt, aux)


def kernel(predictions, targets, alpha):
    b, c = predictions.shape
    assert c == 2 and b % (_NW * _L) == 0
    rows_per_worker = b // _NW
    chunk_rows = min(rows_per_worker, 16384)
    a0 = alpha[0, 0]
    ad = alpha[1, 0] - alpha[0, 0]
    aux = jnp.concatenate([
        jnp.full((_L,), 1.0, jnp.float32) * a0,
        jnp.full((_L,), 1.0, jnp.float32) * ad,
    ])
    rt_zero = jnp.minimum(targets[0], 0).astype(jnp.float32)
    pred_flat = predictions.reshape(-1) + rt_zero
    partials = _focal_partials(pred_flat, targets, aux,
                               rows_per_worker, chunk_rows)
    return jnp.sum(partials)


# speedup vs baseline: 62.8005x; 62.8005x over previous
import functools

import jax
import jax.numpy as jnp
from jax import lax
from jax.experimental import pallas as pl
from jax.experimental.pallas import tpu as pltpu
from jax.experimental.pallas import tpu_sc as plsc

_NC = 2
_NS = 16
_NW = _NC * _NS
_L = 16

_LN2 = 0.6931471805599453
_LOGC = (-1.9316715417209647, 3.498227901209959, -2.420812563219248,
         1.1048082361995168, -0.2806325404497544, 0.030102625011692218)


def _vf(v):
    return jnp.full((_L,), v, jnp.float32)


def _focal_partials(pred_flat, tgt, aux, rows_per_worker, chunk_rows):
    nch = rows_per_worker // chunk_rows
    niter = chunk_rows // _L
    mesh = plsc.VectorSubcoreMesh(core_axis_name="c", subcore_axis_name="s")

    @functools.partial(
        pl.kernel,
        out_type=jax.ShapeDtypeStruct((_NW * _L,), jnp.float32),
        mesh=mesh,
        scratch_types=[
            pltpu.VMEM((2 * chunk_rows,), jnp.float32),
            pltpu.VMEM((2 * chunk_rows,), jnp.float32),
            pltpu.VMEM((chunk_rows,), jnp.int32),
            pltpu.VMEM((chunk_rows,), jnp.int32),
            pltpu.VMEM((2 * _L,), jnp.float32),
            pltpu.VMEM((_L,), jnp.float32),
            pltpu.SemaphoreType.DMA,
            pltpu.SemaphoreType.DMA,
            pltpu.SemaphoreType.DMA,
            pltpu.SemaphoreType.DMA,
        ],
        compiler_params=pltpu.CompilerParams(needs_layout_passes=False),
    )
    def k(pred_hbm, tgt_hbm, aux_hbm, out_hbm,
          pb0, pb1, tb0, tb1, auxv, accv, sp0, sp1, st0, st1):
        wid = lax.axis_index("s") * _NC + lax.axis_index("c")
        pbase = pl.multiple_of(wid * (2 * rows_per_worker), 8)
        tbase = pl.multiple_of(wid * rows_per_worker, 8)

        pltpu.sync_copy(aux_hbm, auxv)
        a0 = auxv[pl.ds(0, _L)]
        ad = auxv[pl.ds(_L, _L)]

        pbufs = (pb0, pb1)
        tbufs = (tb0, tb1)
        psems = (sp0, sp1)
        tsems = (st0, st1)
        copies = [None, None]

        def start(g, b):
            cp = pltpu.async_copy(
                pred_hbm.at[pl.ds(pbase + g * (2 * chunk_rows), 2 * chunk_rows)],
                pbufs[b], psems[b])
            ct = pltpu.async_copy(
                tgt_hbm.at[pl.ds(tbase + g * chunk_rows, chunk_rows)],
                tbufs[b], tsems[b])
            copies[b] = (cp, ct)

        start(0, 0)
        if nch > 1:
            start(1, 1)

        acc = jnp.zeros((_L,), jnp.float32)

        c5 = _vf(_LOGC[5])
        c4 = _vf(_LOGC[4])
        c3 = _vf(_LOGC[3])
        c2 = _vf(_LOGC[2])
        c1 = _vf(_LOGC[1])
        c0 = _vf(_LOGC[0])
        one = _vf(1.0)
        clamp = _vf(80.0)
        ln2 = _vf(_LN2)
        e_bias = _vf(127.0)
        mant_mask = jnp.full((_L,), 0x007FFFFF, jnp.int32)
        one_bits = jnp.full((_L,), 0x3F800000, jnp.int32)
        shift23 = jnp.full((_L,), 23, jnp.int32)

        for g in range(nch):
            b = g & 1
            cp, ct = copies[b]
            cp.wait()
            ct.wait()
            pbuf = pbufs[b]
            tbuf = tbufs[b]

            def body(j, acc, pbuf=pbuf, tbuf=tbuf):
                off0 = (j // 8) * 256 + (j % 8) * _L
                x0 = pbuf[pl.ds(off0, _L)]
                x1 = pbuf[pl.ds(off0 + 128, _L)]
                tv = tbuf[pl.ds(j * _L, _L)]
                tf = tv.astype(jnp.float32)
                s2 = tf + tf - one
                nz = (x0 - x1) * s2
                nz = jnp.minimum(nz, clamp)
                u = jnp.exp(nz)
                w = u + one
                r = one / w
                q = u * r
                sq = q * q
                bits = plsc.bitcast(w, jnp.int32)
                e = lax.shift_right_logical(bits, shift23)
                mbits = jnp.bitwise_or(jnp.bitwise_and(bits, mant_mask),
                                       one_bits)
                mm = plsc.bitcast(mbits, jnp.float32)
                pol = c5
                pol = pol * mm + c4
                pol = pol * mm + c3
                pol = pol * mm + c2
                pol = pol * mm + c1
                pol = pol * mm + c0
                ef = e.astype(jnp.float32) - e_bias
                logw = ef * ln2 + pol
                at = a0 + tf * ad
                acc = acc + at * (sq * logw)
                return acc

            acc = lax.fori_loop(0, niter, body, acc, unroll=8)
            if g + 2 < nch:
                start(g + 2, b)

        accv[...] = acc
        pltpu.sync_copy(accv, out_hbm.at[pl.ds(pl.multiple_of(wid * _L, 8), _L)])

    return k(pred_flat, tgt, aux)


def kernel(predictions, targets, alpha):
    b, c = predictions.shape
    assert c == 2 and b % (_NW * _L) == 0
    rows_per_worker = b // _NW
    chunk_rows = min(rows_per_worker, 16384)
    a0 = alpha[0, 0]
    ad = alpha[1, 0] - alpha[0, 0]
    aux = jnp.concatenate([
        jnp.full((_L,), 1.0, jnp.float32) * a0,
        jnp.full((_L,), 1.0, jnp.float32) * ad,
    ])
    pred_flat = predictions.reshape(-1, 128, 2).transpose(0, 2, 1).reshape(-1)
    partials = _focal_partials(pred_flat, targets, aux,
                               rows_per_worker, chunk_rows)
    return jnp.sum(partials)
